# baseline (speedup 1.0000x reference)
import functools

import jax
import jax.numpy as jnp
from jax import lax
from jax.experimental import pallas as pl
from jax.experimental.pallas import tpu as pltpu
from jax.experimental.pallas import tpu_sc as plsc

D = 64
N = 4096 * 200
L = 16
NC, NS = 2, 16
NW = NC * NS
RPW = N // NW
C = 128
NCHUNK = RPW // C


def _rsqrt(v):
    i = lax.bitcast_convert_type(v, jnp.int32)
    i = jnp.int32(0x5F3759DF) - lax.shift_right_logical(i, 1)
    y = lax.bitcast_convert_type(i, jnp.float32)
    for _ in range(3):
        y = y * (1.5 - 0.5 * v * y * y)
    return y


@functools.partial(
    pl.kernel,
    mesh=plsc.VectorSubcoreMesh(core_axis_name="c", subcore_axis_name="s"),
    out_type=jax.ShapeDtypeStruct((N, D), jnp.float32),
    scratch_types=[
        pltpu.VMEM((C,), jnp.int32),
        pltpu.VMEM((C, D), jnp.float32),
        pltpu.VMEM((D,), jnp.float32),
        pltpu.VMEM((D,), jnp.float32),
        pltpu.SemaphoreType.DMA,
    ],
    compiler_params=pltpu.CompilerParams(
        needs_layout_passes=False, use_tc_tiling_on_sc=False),
)
def _sc_embed_ln(x_hbm, table_hbm, gamma_hbm, beta_hbm, out_hbm,
                 idx_v, rows_v, gamma_v, beta_v, sem):
    wid = lax.axis_index("s") * NC + lax.axis_index("c")
    base = wid * RPW

    pltpu.sync_copy(gamma_hbm, gamma_v)
    pltpu.sync_copy(beta_hbm, beta_v)
    g = [gamma_v[pl.ds(k * L, L)] for k in range(D // L)]
    b = [beta_v[pl.ds(k * L, L)] for k in range(D // L)]

    def row_body(r, carry):
        v = [rows_v[r, pl.ds(k * L, L)] for k in range(D // L)]
        s = (v[0] + v[1]) + (v[2] + v[3])
        sq = (v[0] * v[0] + v[1] * v[1]) + (v[2] * v[2] + v[3] * v[3])
        tot = jnp.sum(s)
        tot2 = jnp.sum(sq)
        mean = tot * (1.0 / D)
        var = tot2 * (1.0 / D) - mean * mean
        rstd = _rsqrt(var + 1e-5)
        for k in range(D // L):
            rows_v[r, pl.ds(k * L, L)] = (v[k] - mean) * rstd * g[k] + b[k]
        return carry

    def chunk_body(c, carry):
        off = base + c * C
        pltpu.sync_copy(x_hbm.at[pl.ds(off, C)], idx_v)
        pltpu.async_copy(table_hbm.at[idx_v], rows_v, sem).wait()
        lax.fori_loop(0, C, row_body, 0, unroll=4)
        pltpu.sync_copy(rows_v, out_hbm.at[pl.ds(off, C)])
        return carry

    lax.fori_loop(0, NCHUNK, chunk_body, 0)


def kernel(x, table, gamma, beta):
    x_flat = x.reshape(-1).astype(jnp.int32)
    out = _sc_embed_ln(x_flat, table, gamma, beta)
    return out.reshape(x.shape + (D,))


# speedup vs baseline: 1.1986x; 1.1986x over previous
import functools

import jax
import jax.numpy as jnp
from jax import lax
from jax.experimental import pallas as pl
from jax.experimental.pallas import tpu as pltpu
from jax.experimental.pallas import tpu_sc as plsc

D = 64
N = 4096 * 200
L = 16
NC, NS = 2, 16
NW = NC * NS
RPW = N // NW
G = 128
C = 256
KG = C // G
NCHUNK = RPW // C


def _rsqrt(v):
    i = lax.bitcast_convert_type(v, jnp.int32)
    i = jnp.int32(0x5F3759DF) - lax.shift_right_logical(i, 1)
    y = lax.bitcast_convert_type(i, jnp.float32)
    for _ in range(3):
        y = y * (1.5 - 0.5 * v * y * y)
    return y


@functools.partial(
    pl.kernel,
    mesh=plsc.VectorSubcoreMesh(core_axis_name="c", subcore_axis_name="s"),
    out_type=jax.ShapeDtypeStruct((N, D), jnp.float32),
    scratch_types=[
        pltpu.VMEM((RPW // G, G), jnp.int32),
        pltpu.VMEM((C, D), jnp.float32),
        pltpu.VMEM((C, D), jnp.float32),
        pltpu.VMEM((D,), jnp.float32),
        pltpu.VMEM((D,), jnp.float32),
        pltpu.SemaphoreType.DMA,
        pltpu.SemaphoreType.DMA,
    ],
    compiler_params=pltpu.CompilerParams(
        needs_layout_passes=False, use_tc_tiling_on_sc=False),
)
def _sc_embed_ln(x_hbm, table_hbm, gamma_hbm, beta_hbm, out_hbm,
                 idx_v, rows_a, rows_b, gamma_v, beta_v, sem_a, sem_b):
    wid = lax.axis_index("s") * NC + lax.axis_index("c")
    base = wid * RPW

    base_g = pl.multiple_of(wid * (RPW // G), 8)
    pltpu.sync_copy(x_hbm.at[pl.ds(base_g, RPW // G)], idx_v)
    pltpu.sync_copy(gamma_hbm, gamma_v)
    pltpu.sync_copy(beta_hbm, beta_v)
    g = [gamma_v[pl.ds(k * L, L)] for k in range(D // L)]
    b = [beta_v[pl.ds(k * L, L)] for k in range(D // L)]

    bufs = ((rows_a, sem_a), (rows_b, sem_b))

    def fire_gather(k, par):
        rows, sem = bufs[par]
        for j in range(KG):
            pltpu.async_copy(table_hbm.at[idx_v.at[k * KG + j]],
                             rows.at[pl.ds(j * G, G)], sem)

    def wait_gather(k, par):
        rows, sem = bufs[par]
        for j in range(KG):
            pltpu.make_async_copy(table_hbm.at[idx_v.at[k * KG + j]],
                                  rows.at[pl.ds(j * G, G)], sem).wait()

    def normalize(rows):
        def row_body(r, carry):
            v = [rows[r, pl.ds(k * L, L)] for k in range(D // L)]
            s = (v[0] + v[1]) + (v[2] + v[3])
            sq = (v[0] * v[0] + v[1] * v[1]) + (v[2] * v[2] + v[3] * v[3])
            mean = jnp.sum(s) * (1.0 / D)
            var = jnp.sum(sq) * (1.0 / D) - mean * mean
            rstd = _rsqrt(var + 1e-5)
            for k in range(D // L):
                rows[r, pl.ds(k * L, L)] = (v[k] - mean) * rstd * g[k] + b[k]
            return carry

        lax.fori_loop(0, C, row_body, 0, unroll=4)

    fire_gather(0, 0)

    def chunk_body(c, carry):
        for par in range(2):
            k = c + par
            wait_gather(k, par)

            @pl.when(k + 1 < NCHUNK)
            def _():
                fire_gather(k + 1, 1 - par)

            rows, _ = bufs[par]
            normalize(rows)
            pltpu.sync_copy(rows, out_hbm.at[pl.ds(base + k * C, C)])
        return carry

    lax.fori_loop(0, NCHUNK // 2, lambda i, cr: chunk_body(i * 2, cr), 0)


def kernel(x, table, gamma, beta):
    x_flat = x.reshape(N // G, G).astype(jnp.int32)
    out = _sc_embed_ln(x_flat, table, gamma, beta)
    return out.reshape(x.shape + (D,))
